# Initial kernel scaffold; baseline (speedup 1.0000x reference)
import jax
import jax.numpy as jnp
from jax.experimental import pallas as pl


def kernel(u):
    raise NotImplementedError("write your pallas kernel here")


# speedup vs baseline: 1.1560x; 1.1560x over previous
import functools

import jax
import jax.numpy as jnp
from jax import lax
from jax.experimental import pallas as pl
from jax.experimental.pallas import tpu as pltpu
from jax.experimental.pallas import tpu_sc as plsc

B, C, H, W = 8, 96, 224, 224
OH, OW = H // 2, W // 2
N_IMG = B * C
IN_SZ = H * W
OUT_SZ = OH * OW
N_WORKERS = 32
IMG_PER_W = N_IMG // N_WORKERS
LANES = 16
CHUNKS = OW // LANES


def _pool_kernel(x_hbm, out_hbm, in_v, out_v):
    c = lax.axis_index("c")
    s = lax.axis_index("s")
    wid = s * 2 + c

    ecol = lax.iota(jnp.int32, LANES) * 2

    def img_body(i, carry):
        img = wid * IMG_PER_W + i
        pltpu.sync_copy(x_hbm.at[img], in_v)

        def row_body(r, carry2):
            base = r * (2 * W)
            for chunk in range(CHUNKS):
                i00 = base + ecol + (chunk * 2 * LANES)
                v00 = plsc.load_gather(in_v, [i00])
                v01 = plsc.load_gather(in_v, [i00 + 1])
                v10 = plsc.load_gather(in_v, [i00 + W])
                v11 = plsc.load_gather(in_v, [i00 + (W + 1)])
                m = jnp.maximum(jnp.maximum(v00, v01), jnp.maximum(v10, v11))
                out_v[pl.ds(r * OW + chunk * LANES, LANES)] = m
            return carry2

        lax.fori_loop(0, OH, row_body, 0)
        pltpu.sync_copy(out_v, out_hbm.at[img])
        return carry

    lax.fori_loop(0, IMG_PER_W, img_body, 0)


def kernel(u):
    x = u.reshape(N_IMG, IN_SZ)
    mesh = plsc.VectorSubcoreMesh(core_axis_name="c", subcore_axis_name="s")
    run = functools.partial(
        pl.kernel,
        mesh=mesh,
        out_type=jax.ShapeDtypeStruct((N_IMG, OUT_SZ), jnp.float32),
        scratch_types=[
            pltpu.VMEM((IN_SZ,), jnp.float32),
            pltpu.VMEM((OUT_SZ,), jnp.float32),
        ],
        compiler_params=pltpu.CompilerParams(needs_layout_passes=False),
    )(_pool_kernel)
    out = run(x)
    return out.reshape(B, C, OH, OW)


# speedup vs baseline: 1.5375x; 1.3301x over previous
import functools

import jax
import jax.numpy as jnp
from jax import lax
from jax.experimental import pallas as pl
from jax.experimental.pallas import tpu as pltpu
from jax.experimental.pallas import tpu_sc as plsc

B, C, H, W = 8, 96, 224, 224
OH, OW = H // 2, W // 2
N_IMG = B * C
IN_SZ = H * W
OUT_SZ = OH * OW
N_WORKERS = 32
IMG_PER_W = N_IMG // N_WORKERS
LANES = 16
CHUNKS = OW // LANES


def _pool_image(in_v, out_v, chunk_cols):
    @plsc.parallel_loop(0, OH, unroll=2)
    def _row(r):
        base = r * (2 * W)
        for chunk in range(CHUNKS):
            i00 = chunk_cols[chunk] + base
            v00 = plsc.load_gather(in_v, [i00])
            v01 = plsc.load_gather(in_v, [i00 + 1])
            v10 = plsc.load_gather(in_v, [i00 + W])
            v11 = plsc.load_gather(in_v, [i00 + (W + 1)])
            m = jnp.maximum(jnp.maximum(v00, v01), jnp.maximum(v10, v11))
            out_v[pl.ds(r * OW + chunk * LANES, LANES)] = m


def _pool_kernel(x_hbm, out_hbm, in0, in1, out0, out1,
                 sem_in0, sem_in1, sem_out0, sem_out1):
    c = lax.axis_index("c")
    s = lax.axis_index("s")
    wid = s * 2 + c
    first = wid * IMG_PER_W

    ecol = lax.iota(jnp.int32, LANES) * 2
    chunk_cols = [ecol + 2 * LANES * chunk for chunk in range(CHUNKS)]

    pltpu.async_copy(x_hbm.at[first], in0, sem_in0)
    pltpu.async_copy(x_hbm.at[first + 1], in1, sem_in1)

    n_pairs = IMG_PER_W // 2

    def pair_body(k, carry):
        img0 = first + 2 * k

        for in_v, out_v, sem_in, sem_out, img in (
            (in0, out0, sem_in0, sem_out0, img0),
            (in1, out1, sem_in1, sem_out1, img0 + 1),
        ):
            pltpu.make_async_copy(x_hbm.at[0], in_v, sem_in).wait()

            @pl.when(k > 0)
            def _():
                pltpu.make_async_copy(out_v, out_hbm.at[0], sem_out).wait()

            _pool_image(in_v, out_v, chunk_cols)
            pltpu.async_copy(out_v, out_hbm.at[img], sem_out)

            @pl.when(k < n_pairs - 1)
            def _():
                pltpu.async_copy(x_hbm.at[img + 2], in_v, sem_in)

        return carry

    lax.fori_loop(0, n_pairs, pair_body, 0)

    pltpu.make_async_copy(out0, out_hbm.at[0], sem_out0).wait()
    pltpu.make_async_copy(out1, out_hbm.at[0], sem_out1).wait()


def kernel(u):
    x = u.reshape(N_IMG, IN_SZ)
    mesh = plsc.VectorSubcoreMesh(core_axis_name="c", subcore_axis_name="s")
    run = functools.partial(
        pl.kernel,
        mesh=mesh,
        out_type=jax.ShapeDtypeStruct((N_IMG, OUT_SZ), jnp.float32),
        scratch_types=[
            pltpu.VMEM((IN_SZ,), jnp.float32),
            pltpu.VMEM((IN_SZ,), jnp.float32),
            pltpu.VMEM((OUT_SZ,), jnp.float32),
            pltpu.VMEM((OUT_SZ,), jnp.float32),
            pltpu.SemaphoreType.DMA,
            pltpu.SemaphoreType.DMA,
            pltpu.SemaphoreType.DMA,
            pltpu.SemaphoreType.DMA,
        ],
        compiler_params=pltpu.CompilerParams(needs_layout_passes=False),
    )(_pool_kernel)
    out = run(x)
    return out.reshape(B, C, OH, OW)


# speedup vs baseline: 5.0294x; 3.2711x over previous
import functools

import jax
import jax.numpy as jnp
from jax import lax
from jax.experimental import pallas as pl
from jax.experimental.pallas import tpu as pltpu
from jax.experimental.pallas import tpu_sc as plsc

B, C, H, W = 8, 96, 224, 224
OH, OW = H // 2, W // 2
N_IMG = B * C
N_WORKERS = 32
IMG_PER_W = N_IMG // N_WORKERS
LANES = 16
CHUNKS = OW // LANES
BH = H // 2
BOH = BH // 2


def _pool_block(in_v, out_v, even_cols, odd_cols):
    @plsc.parallel_loop(0, BOH, unroll=2)
    def _row(r):
        row0 = jnp.full((LANES,), 2 * r, jnp.int32)
        row1 = row0 + 1
        for chunk in range(CHUNKS):
            ce = even_cols[chunk]
            co = odd_cols[chunk]
            v00 = plsc.load_gather(in_v, [row0, ce])
            v01 = plsc.load_gather(in_v, [row0, co])
            v10 = plsc.load_gather(in_v, [row1, ce])
            v11 = plsc.load_gather(in_v, [row1, co])
            m = jnp.maximum(jnp.maximum(v00, v01), jnp.maximum(v10, v11))
            out_v[r, pl.ds(chunk * LANES, LANES)] = m


def _in_slice(x_hbm, img, hh):
    return x_hbm.at[img, pl.ds(hh * BH, BH), :]


def _out_slice(out_hbm, img, hh):
    return out_hbm.at[img, pl.ds(hh * BOH, BOH), :]


def _pool_kernel(x_hbm, out_hbm, in0, in1, out0, out1,
                 sem_in0, sem_in1, sem_out0, sem_out1):
    c = lax.axis_index("c")
    s = lax.axis_index("s")
    wid = s * 2 + c
    first = wid * IMG_PER_W

    ecol = lax.iota(jnp.int32, LANES) * 2
    even_cols = [ecol + 2 * LANES * chunk for chunk in range(CHUNKS)]
    odd_cols = [e + 1 for e in even_cols]

    pltpu.async_copy(_in_slice(x_hbm, first, 0), in0, sem_in0)
    pltpu.async_copy(_in_slice(x_hbm, first, 1), in1, sem_in1)

    def pair_body(k, carry):
        img = first + k

        for in_v, out_v, sem_in, sem_out, hh in (
            (in0, out0, sem_in0, sem_out0, 0),
            (in1, out1, sem_in1, sem_out1, 1),
        ):
            pltpu.make_async_copy(_in_slice(x_hbm, first, 0), in_v,
                                  sem_in).wait()

            @pl.when(k > 0)
            def _():
                pltpu.make_async_copy(out_v, _out_slice(out_hbm, first, 0),
                                      sem_out).wait()

            _pool_block(in_v, out_v, even_cols, odd_cols)
            pltpu.async_copy(out_v, _out_slice(out_hbm, img, hh), sem_out)

            @pl.when(k < IMG_PER_W - 1)
            def _():
                pltpu.async_copy(_in_slice(x_hbm, img + 1, hh), in_v, sem_in)

        return carry

    lax.fori_loop(0, IMG_PER_W, pair_body, 0)

    pltpu.make_async_copy(out0, _out_slice(out_hbm, first, 0), sem_out0).wait()
    pltpu.make_async_copy(out1, _out_slice(out_hbm, first, 1), sem_out1).wait()


def kernel(u):
    x = u.reshape(N_IMG, H, W)
    mesh = plsc.VectorSubcoreMesh(core_axis_name="c", subcore_axis_name="s")
    run = functools.partial(
        pl.kernel,
        mesh=mesh,
        out_type=jax.ShapeDtypeStruct((N_IMG, OH, OW), jnp.float32),
        scratch_types=[
            pltpu.VMEM((BH, W), jnp.float32),
            pltpu.VMEM((BH, W), jnp.float32),
            pltpu.VMEM((BOH, OW), jnp.float32),
            pltpu.VMEM((BOH, OW), jnp.float32),
            pltpu.SemaphoreType.DMA,
            pltpu.SemaphoreType.DMA,
            pltpu.SemaphoreType.DMA,
            pltpu.SemaphoreType.DMA,
        ],
        compiler_params=pltpu.CompilerParams(needs_layout_passes=False),
    )(_pool_kernel)
    out = run(x)
    return out.reshape(B, C, OH, OW)
